# baseline (speedup 1.0000x reference)
import jax
import jax.numpy as jnp
from jax import lax
from jax.experimental import pallas as pl
from jax.experimental.pallas import tpu as pltpu
from jax.experimental.pallas import tpu_sc as plsc

N = 50000
E = 800000
HIST = 12
PRED = 12

NC = 2
NS = 16
SUB = 128
NSUB = 8
K = SUB * NSUB
NSUB2 = 16
K2 = SUB * NSUB2
EP = 851968
PAD = EP - E - N
NZ = 51200
NP = 50048
RPT = NP // NS
ZB = 136

f32 = jnp.float32
i32 = jnp.int32

def _mesh():
    return plsc.VectorSubcoreMesh(
        core_axis_name="c", subcore_axis_name="s",
        num_cores=NC, num_subcores=NS)


def _deg_body(colp2, wp2, degp, col_v, w_v, zero_v, acc, sem):
    c = lax.axis_index("c")
    s = lax.axis_index("s")

    @pl.when(s == 0)
    def _zero():
        def zf(i, _):
            zero_v[pl.ds(i * 16, 16)] = jnp.zeros((16,), f32)
            return 0
        lax.fori_loop(0, 200, zf, 0)
        for j in range(NS):
            pltpu.sync_copy(zero_v, acc.at[pl.ds(j * 3200, 3200)])

    plsc.subcore_barrier()

    n_chunks = EP // (NC * NS * K)
    base_r0 = (c * NS + s) * (EP // (NC * NS * SUB))

    def chunk(i, _):
        br = base_r0 + i * NSUB
        pltpu.sync_copy(colp2.at[pl.ds(br, NSUB)], col_v)
        pltpu.sync_copy(wp2.at[pl.ds(br, NSUB)], w_v)
        for b in range(NSUB):
            pltpu.sync_copy(w_v.at[b], acc.at[col_v.at[b]], add=True)
        return 0

    lax.fori_loop(0, n_chunks, chunk, 0)
    plsc.subcore_barrier()

    @pl.when(s == 0)
    def _write():
        pltpu.sync_copy(acc, degp.at[pl.ds(c * NZ, NZ)])


def _build_deg():
    return pl.kernel(
        _deg_body,
        out_type=jax.ShapeDtypeStruct((NC * NZ,), f32),
        mesh=_mesh(),
        compiler_params=pltpu.CompilerParams(needs_layout_passes=False, use_tc_tiling_on_sc=False),
        scratch_types=[
            pltpu.VMEM((NSUB, SUB), i32),
            pltpu.VMEM((NSUB, SUB), f32),
            pltpu.VMEM((3200,), f32),
            pltpu.VMEM_SHARED((NZ,), f32),
            pltpu.SemaphoreType.DMA,
        ],
    )


def _norm_body(rowp2, colp2, wp2, dinv, normp2, row_v, col_v, w_v, dinv_v, sem):
    c = lax.axis_index("c")
    s = lax.axis_index("s")
    pltpu.sync_copy(dinv, dinv_v)

    n_chunks = EP // (NC * NS * K)
    base_r0 = (c * NS + s) * (EP // (NC * NS * SUB))

    def chunk(i, _):
        br = base_r0 + i * NSUB
        pltpu.sync_copy(rowp2.at[pl.ds(br, NSUB)], row_v)
        pltpu.sync_copy(colp2.at[pl.ds(br, NSUB)], col_v)
        pltpu.sync_copy(wp2.at[pl.ds(br, NSUB)], w_v)
        for b in range(NSUB):
            def g16(g, _, b=b):
                sl = pl.ds(g * 16, 16)
                r16 = row_v[b, sl]
                c16 = col_v[b, sl]
                dr = plsc.load_gather(dinv_v, [r16 >> 7, r16 & 127])
                dc = plsc.load_gather(dinv_v, [c16 >> 7, c16 & 127])
                w_v[b, sl] = dr * w_v[b, sl] * dc
                return 0
            lax.fori_loop(0, SUB // 16, g16, 0)
        pltpu.sync_copy(w_v, normp2.at[pl.ds(br, NSUB)])
        return 0

    lax.fori_loop(0, n_chunks, chunk, 0)


def _build_norm():
    return pl.kernel(
        _norm_body,
        out_type=jax.ShapeDtypeStruct((EP // SUB, SUB), f32),
        mesh=_mesh(),
        compiler_params=pltpu.CompilerParams(needs_layout_passes=False, use_tc_tiling_on_sc=False),
        scratch_types=[
            pltpu.VMEM((NSUB, SUB), i32),
            pltpu.VMEM((NSUB, SUB), i32),
            pltpu.VMEM((NSUB, SUB), f32),
            pltpu.VMEM((NZ // 128, 128), f32),
            pltpu.SemaphoreType.DMA,
        ],
    )


def _make_spmm(feature_split, boff=0):
    H = 16
    if feature_split:
        n_chunks = EP // (NS * K2)
    else:
        n_chunks = EP // (NC * NS * K2)

    qb = boff // N

    def body(table, rows4, colp2, normp2, out,
             row_v, col_v, norm_v, rows_v, msg_v, zero_v, acc, sem):
        c = lax.axis_index("c")
        s = lax.axis_index("s")

        def zf(i, _):
            zero_v[i, pl.ds(0, 16)] = jnp.zeros((16,), f32)
            return 0
        lax.fori_loop(0, ZB, zf, 0)
        for j in range(RPT // ZB):
            pltpu.sync_copy(zero_v, acc.at[pl.ds(s * RPT + j * ZB, ZB)])
        plsc.subcore_barrier()

        if feature_split:
            base_r0 = s * (EP // (NS * SUB))
        else:
            base_r0 = (c * NS + s) * (EP // (NC * NS * SUB))

        def chunk(i, _):
            br = base_r0 + i * NSUB2
            pltpu.sync_copy(rows4.at[qb + c, pl.ds(br, NSUB2)], row_v)
            pltpu.sync_copy(colp2.at[pl.ds(br, NSUB2)], col_v)
            pltpu.sync_copy(normp2.at[pl.ds(br, NSUB2)], norm_v)
            descs = [
                pltpu.async_copy(table.at[row_v.at[b]],
                                 rows_v.at[pl.ds(b * SUB, SUB)], sem)
                for b in range(NSUB2)
            ]
            for d in descs:
                d.wait()
            for b in range(NSUB2):
                @plsc.parallel_loop(0, SUB // 16, 1, unroll=4)
                def _scale(g, b=b):
                    n16 = norm_v[b, pl.ds(g * 16, 16)]
                    r16 = lax.iota(i32, 16) + (b * SUB + g * 16)
                    for t in range(H):
                        cvec = jnp.full((16,), t, i32)
                        v = plsc.load_gather(rows_v, [r16, cvec])
                        plsc.store_scatter(msg_v, [r16, cvec], v * n16)
            for b in range(NSUB2):
                pltpu.sync_copy(msg_v.at[pl.ds(b * SUB, SUB)],
                                acc.at[col_v.at[b]], add=True)
            return 0

        lax.fori_loop(0, n_chunks, chunk, 0)
        plsc.subcore_barrier()
        pltpu.sync_copy(acc.at[pl.ds(s * RPT, RPT)],
                        out.at[pl.ds(c * NP + s * RPT, RPT)])

    return pl.kernel(
        body,
        out_type=jax.ShapeDtypeStruct((NC * NP, H), f32),
        mesh=_mesh(),
        compiler_params=pltpu.CompilerParams(needs_layout_passes=False, use_tc_tiling_on_sc=False),
        scratch_types=[
            pltpu.VMEM((NSUB2, SUB), i32),
            pltpu.VMEM((NSUB2, SUB), i32),
            pltpu.VMEM((NSUB2, SUB), f32),
            pltpu.VMEM((K2, H), f32),
            pltpu.VMEM((K2, H), f32),
            pltpu.VMEM((ZB, H), f32),
            pltpu.VMEM_SHARED((NP, H), f32),
            pltpu.SemaphoreType.DMA,
        ],
    )


_sc_cache = {}


def _sc(name):
    if name not in _sc_cache:
        _sc_cache["deg"] = _build_deg()
        _sc_cache["norm"] = _build_norm()
        _sc_cache["spmm16f"] = _make_spmm(True)
        _sc_cache["spmm16fb"] = _make_spmm(True, boff=2 * N)
        _sc_cache["spmm16e"] = _make_spmm(False)
    return _sc_cache[name]


BN = 2000
G = N // BN


def _spec(kind, d=0):
    if kind == "r":
        return pl.BlockSpec((BN, d), lambda i: (i, 0))
    if kind == "h":
        return pl.BlockSpec((2, BN, d), lambda i: (0, i, 0))
    if kind == "h4":
        return pl.BlockSpec((4, BN, d), lambda i: (0, i, 0))
    if kind == "w":
        return pl.BlockSpec(d, lambda i: tuple(0 for _ in d))
    raise ValueError(kind)


def _dinv_body(degp, dinv):
    dp = degp[...]
    dinv[...] = lax.rsqrt(dp[0] + dp[1])


def _dinv_call(degp, interpret=False):
    out = pl.pallas_call(
        _dinv_body,
        out_shape=jax.ShapeDtypeStruct((20, 2560), f32),
        interpret=interpret,
    )(degp.reshape(2, 20, 2560))
    return out.reshape(NZ)


def _tc(body, ins, in_specs, out_shapes, out_specs, interpret=False):
    return pl.pallas_call(
        body,
        grid=(G,),
        in_specs=in_specs,
        out_specs=out_specs,
        out_shape=out_shapes,
        interpret=interpret,
    )(*ins)


def _tc1_body(x, y, we1, h1):
    theta = jnp.concatenate([x[...], y[...]], axis=1)
    h = jnp.dot(theta, we1[...], preferred_element_type=f32)
    h1[0] = h[:, :16]
    h1[1] = h[:, 16:]


def _tc2_body(s1, b1, we2, h2):
    theta = jnp.maximum(
        jnp.concatenate([s1[0], s1[1]], axis=1) + b1[...], 0.0)
    h = jnp.dot(theta, we2[...], preferred_element_type=f32)
    for q in range(4):
        h2[q] = h[:, q * 16:(q + 1) * 16]


def _tc3_body(s2a, s2b, b2, wml, h3):
    theta = jnp.maximum(
        jnp.concatenate([s2a[0], s2a[1], s2b[0], s2b[1]], axis=1)
        + b2[...], 0.0)
    h = jnp.dot(theta, wml[...], preferred_element_type=f32)
    h3[0] = h
    h3[1] = h


def _tc4_body(s3, x, eps, bml, wg0, ws0, h4, ys0):
    ml = s3[0] + s3[1] + bml[...]
    mu = ml[:, 0:1]
    logvar = ml[:, 1:2]
    z = mu + eps[...] * jnp.exp(0.5 * logvar)
    recon = jnp.concatenate([z, x[...]], axis=1)
    h = jnp.dot(recon, wg0[...], preferred_element_type=f32)
    h4[0] = h[:, :16]
    h4[1] = h[:, 16:]
    ys0[...] = jnp.dot(recon, ws0[...], preferred_element_type=f32)


def _tc5_body(s4, ys0, x, bg0, wg1, ws1, h5, ys1):
    g = jnp.maximum(jnp.concatenate([s4[0], s4[1]], axis=1) + bg0[...], 0.0)
    yh = g + ys0[...]
    recon = jnp.concatenate([yh, x[...]], axis=1)
    h = jnp.dot(recon, wg1[...], preferred_element_type=f32)
    for q in range(4):
        h5[q] = h[:, q * 16:(q + 1) * 16]
    ys1[...] = jnp.dot(recon, ws1[...], preferred_element_type=f32)


def _tc6_body(s5a, s5b, ys1, x, bg1, wg2, ws2, h6, ys2):
    g = jnp.maximum(
        jnp.concatenate([s5a[0], s5a[1], s5b[0], s5b[1]], axis=1)
        + bg1[...], 0.0)
    yh = g + ys1[...]
    recon = jnp.concatenate([yh, x[...]], axis=1)
    h = jnp.dot(recon, wg2[...], preferred_element_type=f32)
    h6[0] = h
    h6[1] = h
    ys2[...] = jnp.dot(recon, ws2[...], preferred_element_type=f32)


def _tc7_body(s6, ys2, bg2, out):
    g = jnp.maximum(s6[0][:, :PRED] + s6[1][:, :PRED] + bg2[...], 0.0)
    out[...] = g + ys2[...]


def _run(x, y, edge_idx, edge_wt, params, interpret=False,
         deg_call=None, norm_call=None, spmm16f=None, spmm16fb=None,
         spmm16e=None):
    deg_call = deg_call or _sc("deg")
    norm_call = norm_call or _sc("norm")
    spmm16f = spmm16f or _sc("spmm16f")
    spmm16fb = spmm16fb or _sc("spmm16fb")
    spmm16e = spmm16e or _sc("spmm16e")
    p = params

    row = edge_idx[0]
    col = edge_idx[1]
    loops = jnp.arange(N, dtype=i32)
    padi = (jnp.arange(PAD, dtype=i32) * 11) % N
    rowp = jnp.concatenate([row, loops, padi]).reshape(EP // SUB, SUB)
    rows4 = rowp[None, :, :] + (jnp.arange(4, dtype=i32)[:, None, None] * N)
    colp = jnp.concatenate([col, loops, padi]).reshape(EP // SUB, SUB)
    wp = jnp.concatenate([
        edge_wt, jnp.full((N,), 2.0, f32), jnp.zeros((PAD,), f32)
    ]).reshape(EP // SUB, SUB)

    eps = jax.random.uniform(jax.random.key(42), (N, 1), dtype=f32)

    wml = jnp.pad(jnp.concatenate([p['W_mu'], p['W_var']], axis=1),
                  ((0, 0), (0, 14)))
    bml = jnp.pad(jnp.concatenate([p['b_mu'], p['b_var']]), (0, 14))
    wg2 = jnp.pad(p['W_g2'], ((0, 0), (0, 16 - PRED)))

    def b2d(b):
        return b.reshape(1, -1)

    degp = deg_call(colp, wp)
    dinv = _dinv_call(degp, interpret)
    h1 = _tc(
        _tc1_body,
        (x, y, p['W_e1']),
        [_spec("r", HIST), _spec("r", PRED), _spec("w", (24, 32))],
        jax.ShapeDtypeStruct((2, N, 16), f32),
        _spec("h", 16),
        interpret,
    )
    normp = norm_call(rowp, colp, wp, dinv.reshape(NZ // 128, 128))

    s1 = spmm16f(h1.reshape(2 * N, 16), rows4, colp, normp)
    h2 = _tc(
        _tc2_body,
        (s1.reshape(2, NP, 16), b2d(p['b_e1']), p['W_e2']),
        [_spec("h", 16), _spec("w", (1, 32)), _spec("w", (32, 64))],
        jax.ShapeDtypeStruct((4, N, 16), f32),
        _spec("h4", 16),
        interpret,
    )
    h2v = h2.reshape(4 * N, 16)
    s2a = spmm16f(h2v, rows4, colp, normp)
    s2b = spmm16fb(h2v, rows4, colp, normp)
    h3 = _tc(
        _tc3_body,
        (s2a.reshape(2, NP, 16), s2b.reshape(2, NP, 16), b2d(p['b_e2']), wml),
        [_spec("h", 16), _spec("h", 16), _spec("w", (1, 64)),
         _spec("w", (64, 16))],
        jax.ShapeDtypeStruct((2, N, 16), f32),
        _spec("h", 16),
        interpret,
    )
    s3 = spmm16e(h3.reshape(2 * N, 16), rows4, colp, normp)
    h4, ys0 = _tc(
        _tc4_body,
        (s3.reshape(2, NP, 16), x, eps, b2d(bml), p['W_g0'], p['W_s0']),
        [_spec("h", 16), _spec("r", HIST), _spec("r", 1), _spec("w", (1, 16)),
         _spec("w", (13, 32)), _spec("w", (13, 32))],
        (jax.ShapeDtypeStruct((2, N, 16), f32),
         jax.ShapeDtypeStruct((N, 32), f32)),
        (_spec("h", 16), _spec("r", 32)),
        interpret,
    )
    s4 = spmm16f(h4.reshape(2 * N, 16), rows4, colp, normp)
    h5, ys1 = _tc(
        _tc5_body,
        (s4.reshape(2, NP, 16), ys0, x, b2d(p['b_g0']), p['W_g1'], p['W_s1']),
        [_spec("h", 16), _spec("r", 32), _spec("r", HIST), _spec("w", (1, 32)),
         _spec("w", (44, 64)), _spec("w", (44, 64))],
        (jax.ShapeDtypeStruct((4, N, 16), f32),
         jax.ShapeDtypeStruct((N, 64), f32)),
        (_spec("h4", 16), _spec("r", 64)),
        interpret,
    )
    h5v = h5.reshape(4 * N, 16)
    s5a = spmm16f(h5v, rows4, colp, normp)
    s5b = spmm16fb(h5v, rows4, colp, normp)
    h6, ys2 = _tc(
        _tc6_body,
        (s5a.reshape(2, NP, 16), s5b.reshape(2, NP, 16), ys1, x,
         b2d(p['b_g1']), wg2, p['W_s2']),
        [_spec("h", 16), _spec("h", 16), _spec("r", 64), _spec("r", HIST),
         _spec("w", (1, 64)), _spec("w", (76, 16)), _spec("w", (76, PRED))],
        (jax.ShapeDtypeStruct((2, N, 16), f32),
         jax.ShapeDtypeStruct((N, PRED), f32)),
        (_spec("h", 16), _spec("r", PRED)),
        interpret,
    )
    s6 = spmm16e(h6.reshape(2 * N, 16), rows4, colp, normp)
    out = _tc(
        _tc7_body,
        (s6.reshape(2, NP, 16), ys2, b2d(p['b_g2'])),
        [_spec("h", 16), _spec("r", PRED), _spec("w", (1, PRED))],
        jax.ShapeDtypeStruct((N, PRED), f32),
        _spec("r", PRED),
        interpret,
    )
    return out


def kernel(x, y, edge_idx, edge_wt, params):
    return _run(x, y, edge_idx, edge_wt, params)


# speedup vs baseline: 1.1193x; 1.1193x over previous
import jax
import jax.numpy as jnp
from jax import lax
from jax.experimental import pallas as pl
from jax.experimental.pallas import tpu as pltpu
from jax.experimental.pallas import tpu_sc as plsc

N = 50000
E = 800000
HIST = 12
PRED = 12

NC = 2
NS = 16
SUB = 128
NSUB = 8
K = SUB * NSUB
NSUB2 = 16
K2 = SUB * NSUB2
EP = 851968
PAD = EP - E - N
NZ = 51200
NP = 50048
RPT = NP // NS
ZB = 136

f32 = jnp.float32
i32 = jnp.int32

def _mesh():
    return plsc.VectorSubcoreMesh(
        core_axis_name="c", subcore_axis_name="s",
        num_cores=NC, num_subcores=NS)


def _deg_body(colp2, wp2, degp, col_v, w_v, zero_v, acc, sem):
    c = lax.axis_index("c")
    s = lax.axis_index("s")

    @pl.when(s == 0)
    def _zero():
        def zf(i, _):
            zero_v[pl.ds(i * 16, 16)] = jnp.zeros((16,), f32)
            return 0
        lax.fori_loop(0, 200, zf, 0)
        for j in range(NS):
            pltpu.sync_copy(zero_v, acc.at[pl.ds(j * 3200, 3200)])

    plsc.subcore_barrier()

    n_chunks = EP // (NC * NS * K)
    base_r0 = (c * NS + s) * (EP // (NC * NS * SUB))

    def chunk(i, _):
        br = base_r0 + i * NSUB
        pltpu.sync_copy(colp2.at[pl.ds(br, NSUB)], col_v)
        pltpu.sync_copy(wp2.at[pl.ds(br, NSUB)], w_v)
        for b in range(NSUB):
            pltpu.sync_copy(w_v.at[b], acc.at[col_v.at[b]], add=True)
        return 0

    lax.fori_loop(0, n_chunks, chunk, 0)
    plsc.subcore_barrier()

    @pl.when(s == 0)
    def _write():
        pltpu.sync_copy(acc, degp.at[pl.ds(c * NZ, NZ)])


def _build_deg():
    return pl.kernel(
        _deg_body,
        out_type=jax.ShapeDtypeStruct((NC * NZ,), f32),
        mesh=_mesh(),
        compiler_params=pltpu.CompilerParams(needs_layout_passes=False, use_tc_tiling_on_sc=False),
        scratch_types=[
            pltpu.VMEM((NSUB, SUB), i32),
            pltpu.VMEM((NSUB, SUB), f32),
            pltpu.VMEM((3200,), f32),
            pltpu.VMEM_SHARED((NZ,), f32),
            pltpu.SemaphoreType.DMA,
        ],
    )


def _norm_body(rowp2, colp2, wp2, dinv, normp2, row_v, col_v, w_v, dinv_v, sem):
    c = lax.axis_index("c")
    s = lax.axis_index("s")
    pltpu.sync_copy(dinv, dinv_v)

    n_chunks = EP // (NC * NS * K)
    base_r0 = (c * NS + s) * (EP // (NC * NS * SUB))

    def chunk(i, _):
        br = base_r0 + i * NSUB
        pltpu.sync_copy(rowp2.at[pl.ds(br, NSUB)], row_v)
        pltpu.sync_copy(colp2.at[pl.ds(br, NSUB)], col_v)
        pltpu.sync_copy(wp2.at[pl.ds(br, NSUB)], w_v)
        for b in range(NSUB):
            def g16(g, _, b=b):
                sl = pl.ds(g * 16, 16)
                r16 = row_v[b, sl]
                c16 = col_v[b, sl]
                dr = plsc.load_gather(dinv_v, [r16 >> 7, r16 & 127])
                dc = plsc.load_gather(dinv_v, [c16 >> 7, c16 & 127])
                w_v[b, sl] = dr * w_v[b, sl] * dc
                return 0
            lax.fori_loop(0, SUB // 16, g16, 0)
        pltpu.sync_copy(w_v, normp2.at[pl.ds(br, NSUB)])
        return 0

    lax.fori_loop(0, n_chunks, chunk, 0)


def _build_norm():
    return pl.kernel(
        _norm_body,
        out_type=jax.ShapeDtypeStruct((EP // SUB, SUB), f32),
        mesh=_mesh(),
        compiler_params=pltpu.CompilerParams(needs_layout_passes=False, use_tc_tiling_on_sc=False),
        scratch_types=[
            pltpu.VMEM((NSUB, SUB), i32),
            pltpu.VMEM((NSUB, SUB), i32),
            pltpu.VMEM((NSUB, SUB), f32),
            pltpu.VMEM((NZ // 128, 128), f32),
            pltpu.SemaphoreType.DMA,
        ],
    )


def _make_spmm(feature_split, boff=0):
    H = 16
    if feature_split:
        n_chunks = EP // (NS * K2)
    else:
        n_chunks = EP // (NC * NS * K2)

    qb = boff // N

    def body(table, rows4, colp2, normp2, out,
             row_v, col_v, norm_v, rows_v, msg_v, zero_v, acc, sem):
        c = lax.axis_index("c")
        s = lax.axis_index("s")

        def zf(i, _):
            zero_v[i, pl.ds(0, 16)] = jnp.zeros((16,), f32)
            return 0
        lax.fori_loop(0, ZB, zf, 0)
        for j in range(RPT // ZB):
            pltpu.sync_copy(zero_v, acc.at[pl.ds(s * RPT + j * ZB, ZB)])
        plsc.subcore_barrier()

        if feature_split:
            base_r0 = s * (EP // (NS * SUB))
        else:
            base_r0 = (c * NS + s) * (EP // (NC * NS * SUB))

        def chunk(i, _):
            br = base_r0 + i * NSUB2
            pltpu.sync_copy(rows4.at[qb + c, pl.ds(br, NSUB2)], row_v)
            pltpu.sync_copy(colp2.at[pl.ds(br, NSUB2)], col_v)
            pltpu.sync_copy(normp2.at[pl.ds(br, NSUB2)], norm_v)
            descs = [
                pltpu.async_copy(table.at[row_v.at[b]],
                                 rows_v.at[pl.ds(b * SUB, SUB)], sem)
                for b in range(NSUB2)
            ]
            for d in descs:
                d.wait()
            for b in range(NSUB2):
                @plsc.parallel_loop(0, SUB // 16, 1, unroll=2)
                def _scale(g, b=b):
                    n16 = norm_v[b, pl.ds(g * 16, 16)]
                    r16 = lax.iota(i32, 16) + (b * SUB + g * 16)
                    for t in range(H):
                        cvec = jnp.full((16,), t, i32)
                        v = plsc.load_gather(rows_v, [r16, cvec])
                        plsc.store_scatter(msg_v, [r16, cvec], v * n16)
            for b in range(NSUB2):
                pltpu.sync_copy(msg_v.at[pl.ds(b * SUB, SUB)],
                                acc.at[col_v.at[b]], add=True)
            return 0

        lax.fori_loop(0, n_chunks, chunk, 0)
        plsc.subcore_barrier()
        pltpu.sync_copy(acc.at[pl.ds(s * RPT, RPT)],
                        out.at[pl.ds(c * NP + s * RPT, RPT)])

    return pl.kernel(
        body,
        out_type=jax.ShapeDtypeStruct((NC * NP, H), f32),
        mesh=_mesh(),
        compiler_params=pltpu.CompilerParams(needs_layout_passes=False, use_tc_tiling_on_sc=False),
        scratch_types=[
            pltpu.VMEM((NSUB2, SUB), i32),
            pltpu.VMEM((NSUB2, SUB), i32),
            pltpu.VMEM((NSUB2, SUB), f32),
            pltpu.VMEM((K2, H), f32),
            pltpu.VMEM((K2, H), f32),
            pltpu.VMEM((ZB, H), f32),
            pltpu.VMEM_SHARED((NP, H), f32),
            pltpu.SemaphoreType.DMA,
        ],
    )


_sc_cache = {}


def _sc(name):
    if name not in _sc_cache:
        _sc_cache["deg"] = _build_deg()
        _sc_cache["norm"] = _build_norm()
        _sc_cache["spmm16f"] = _make_spmm(True)
        _sc_cache["spmm16fb"] = _make_spmm(True, boff=2 * N)
        _sc_cache["spmm16e"] = _make_spmm(False)
    return _sc_cache[name]


BN = 2000
G = N // BN


def _spec(kind, d=0):
    if kind == "r":
        return pl.BlockSpec((BN, d), lambda i: (i, 0))
    if kind == "h":
        return pl.BlockSpec((2, BN, d), lambda i: (0, i, 0))
    if kind == "h4":
        return pl.BlockSpec((4, BN, d), lambda i: (0, i, 0))
    if kind == "w":
        return pl.BlockSpec(d, lambda i: tuple(0 for _ in d))
    raise ValueError(kind)


def _dinv_body(degp, dinv):
    dp = degp[...]
    dinv[...] = lax.rsqrt(dp[0] + dp[1])


def _dinv_call(degp, interpret=False):
    out = pl.pallas_call(
        _dinv_body,
        out_shape=jax.ShapeDtypeStruct((20, 2560), f32),
        interpret=interpret,
    )(degp.reshape(2, 20, 2560))
    return out.reshape(NZ)


def _tc(body, ins, in_specs, out_shapes, out_specs, interpret=False):
    return pl.pallas_call(
        body,
        grid=(G,),
        in_specs=in_specs,
        out_specs=out_specs,
        out_shape=out_shapes,
        interpret=interpret,
    )(*ins)


def _tc1_body(x, y, we1, h1):
    theta = jnp.concatenate([x[...], y[...]], axis=1)
    h = jnp.dot(theta, we1[...], preferred_element_type=f32)
    h1[0] = h[:, :16]
    h1[1] = h[:, 16:]


def _tc2_body(s1, b1, we2, h2):
    theta = jnp.maximum(
        jnp.concatenate([s1[0], s1[1]], axis=1) + b1[...], 0.0)
    h = jnp.dot(theta, we2[...], preferred_element_type=f32)
    for q in range(4):
        h2[q] = h[:, q * 16:(q + 1) * 16]


def _tc3_body(s2a, s2b, b2, wml, h3):
    theta = jnp.maximum(
        jnp.concatenate([s2a[0], s2a[1], s2b[0], s2b[1]], axis=1)
        + b2[...], 0.0)
    h = jnp.dot(theta, wml[...], preferred_element_type=f32)
    h3[0] = h
    h3[1] = h


def _tc4_body(s3, x, eps, bml, wg0, ws0, h4, ys0):
    ml = s3[0] + s3[1] + bml[...]
    mu = ml[:, 0:1]
    logvar = ml[:, 1:2]
    z = mu + eps[...] * jnp.exp(0.5 * logvar)
    recon = jnp.concatenate([z, x[...]], axis=1)
    h = jnp.dot(recon, wg0[...], preferred_element_type=f32)
    h4[0] = h[:, :16]
    h4[1] = h[:, 16:]
    ys0[...] = jnp.dot(recon, ws0[...], preferred_element_type=f32)


def _tc5_body(s4, ys0, x, bg0, wg1, ws1, h5, ys1):
    g = jnp.maximum(jnp.concatenate([s4[0], s4[1]], axis=1) + bg0[...], 0.0)
    yh = g + ys0[...]
    recon = jnp.concatenate([yh, x[...]], axis=1)
    h = jnp.dot(recon, wg1[...], preferred_element_type=f32)
    for q in range(4):
        h5[q] = h[:, q * 16:(q + 1) * 16]
    ys1[...] = jnp.dot(recon, ws1[...], preferred_element_type=f32)


def _tc6_body(s5a, s5b, ys1, x, bg1, wg2, ws2, h6, ys2):
    g = jnp.maximum(
        jnp.concatenate([s5a[0], s5a[1], s5b[0], s5b[1]], axis=1)
        + bg1[...], 0.0)
    yh = g + ys1[...]
    recon = jnp.concatenate([yh, x[...]], axis=1)
    h = jnp.dot(recon, wg2[...], preferred_element_type=f32)
    h6[0] = h
    h6[1] = h
    ys2[...] = jnp.dot(recon, ws2[...], preferred_element_type=f32)


def _tc7_body(s6, ys2, bg2, out):
    g = jnp.maximum(s6[0][:, :PRED] + s6[1][:, :PRED] + bg2[...], 0.0)
    out[...] = g + ys2[...]


def _run(x, y, edge_idx, edge_wt, params, interpret=False,
         deg_call=None, norm_call=None, spmm16f=None, spmm16fb=None,
         spmm16e=None):
    deg_call = deg_call or _sc("deg")
    norm_call = norm_call or _sc("norm")
    spmm16f = spmm16f or _sc("spmm16f")
    spmm16fb = spmm16fb or _sc("spmm16fb")
    spmm16e = spmm16e or _sc("spmm16e")
    p = params

    row = edge_idx[0]
    col = edge_idx[1]
    loops = jnp.arange(N, dtype=i32)
    padi = (jnp.arange(PAD, dtype=i32) * 11) % N
    rowp = jnp.concatenate([row, loops, padi]).reshape(EP // SUB, SUB)
    rows4 = rowp[None, :, :] + (jnp.arange(4, dtype=i32)[:, None, None] * N)
    colp = jnp.concatenate([col, loops, padi]).reshape(EP // SUB, SUB)
    wp = jnp.concatenate([
        edge_wt, jnp.full((N,), 2.0, f32), jnp.zeros((PAD,), f32)
    ]).reshape(EP // SUB, SUB)

    eps = jax.random.uniform(jax.random.key(42), (N, 1), dtype=f32)

    wml = jnp.pad(jnp.concatenate([p['W_mu'], p['W_var']], axis=1),
                  ((0, 0), (0, 14)))
    bml = jnp.pad(jnp.concatenate([p['b_mu'], p['b_var']]), (0, 14))
    wg2 = jnp.pad(p['W_g2'], ((0, 0), (0, 16 - PRED)))

    def b2d(b):
        return b.reshape(1, -1)

    degp = deg_call(colp, wp)
    dinv = _dinv_call(degp, interpret)
    h1 = _tc(
        _tc1_body,
        (x, y, p['W_e1']),
        [_spec("r", HIST), _spec("r", PRED), _spec("w", (24, 32))],
        jax.ShapeDtypeStruct((2, N, 16), f32),
        _spec("h", 16),
        interpret,
    )
    normp = norm_call(rowp, colp, wp, dinv.reshape(NZ // 128, 128))

    s1 = spmm16f(h1.reshape(2 * N, 16), rows4, colp, normp)
    h2 = _tc(
        _tc2_body,
        (s1.reshape(2, NP, 16), b2d(p['b_e1']), p['W_e2']),
        [_spec("h", 16), _spec("w", (1, 32)), _spec("w", (32, 64))],
        jax.ShapeDtypeStruct((4, N, 16), f32),
        _spec("h4", 16),
        interpret,
    )
    h2v = h2.reshape(4 * N, 16)
    s2a = spmm16f(h2v, rows4, colp, normp)
    s2b = spmm16fb(h2v, rows4, colp, normp)
    h3 = _tc(
        _tc3_body,
        (s2a.reshape(2, NP, 16), s2b.reshape(2, NP, 16), b2d(p['b_e2']), wml),
        [_spec("h", 16), _spec("h", 16), _spec("w", (1, 64)),
         _spec("w", (64, 16))],
        jax.ShapeDtypeStruct((2, N, 16), f32),
        _spec("h", 16),
        interpret,
    )
    s3 = spmm16e(h3.reshape(2 * N, 16), rows4, colp, normp)
    h4, ys0 = _tc(
        _tc4_body,
        (s3.reshape(2, NP, 16), x, eps, b2d(bml), p['W_g0'], p['W_s0']),
        [_spec("h", 16), _spec("r", HIST), _spec("r", 1), _spec("w", (1, 16)),
         _spec("w", (13, 32)), _spec("w", (13, 32))],
        (jax.ShapeDtypeStruct((2, N, 16), f32),
         jax.ShapeDtypeStruct((N, 32), f32)),
        (_spec("h", 16), _spec("r", 32)),
        interpret,
    )
    s4 = spmm16f(h4.reshape(2 * N, 16), rows4, colp, normp)
    h5, ys1 = _tc(
        _tc5_body,
        (s4.reshape(2, NP, 16), ys0, x, b2d(p['b_g0']), p['W_g1'], p['W_s1']),
        [_spec("h", 16), _spec("r", 32), _spec("r", HIST), _spec("w", (1, 32)),
         _spec("w", (44, 64)), _spec("w", (44, 64))],
        (jax.ShapeDtypeStruct((4, N, 16), f32),
         jax.ShapeDtypeStruct((N, 64), f32)),
        (_spec("h4", 16), _spec("r", 64)),
        interpret,
    )
    h5v = h5.reshape(4 * N, 16)
    s5a = spmm16f(h5v, rows4, colp, normp)
    s5b = spmm16fb(h5v, rows4, colp, normp)
    h6, ys2 = _tc(
        _tc6_body,
        (s5a.reshape(2, NP, 16), s5b.reshape(2, NP, 16), ys1, x,
         b2d(p['b_g1']), wg2, p['W_s2']),
        [_spec("h", 16), _spec("h", 16), _spec("r", 64), _spec("r", HIST),
         _spec("w", (1, 64)), _spec("w", (76, 16)), _spec("w", (76, PRED))],
        (jax.ShapeDtypeStruct((2, N, 16), f32),
         jax.ShapeDtypeStruct((N, PRED), f32)),
        (_spec("h", 16), _spec("r", PRED)),
        interpret,
    )
    s6 = spmm16e(h6.reshape(2 * N, 16), rows4, colp, normp)
    out = _tc(
        _tc7_body,
        (s6.reshape(2, NP, 16), ys2, b2d(p['b_g2'])),
        [_spec("h", 16), _spec("r", PRED), _spec("w", (1, PRED))],
        jax.ShapeDtypeStruct((N, PRED), f32),
        _spec("r", PRED),
        interpret,
    )
    return out


def kernel(x, y, edge_idx, edge_wt, params):
    return _run(x, y, edge_idx, edge_wt, params)


# speedup vs baseline: 1.1804x; 1.0546x over previous
import jax
import jax.numpy as jnp
from jax import lax
from jax.experimental import pallas as pl
from jax.experimental.pallas import tpu as pltpu
from jax.experimental.pallas import tpu_sc as plsc

N = 50000
E = 800000
HIST = 12
PRED = 12

NC = 2
NS = 16
SUB = 128
NSUB = 8
K = SUB * NSUB
NSUB2 = 8
K2 = SUB * NSUB2
EP = 851968
PAD = EP - E - N
NZ = 51200
NP = 50048
RPT = NP // NS
ZB = 136

f32 = jnp.float32
i32 = jnp.int32

def _mesh():
    return plsc.VectorSubcoreMesh(
        core_axis_name="c", subcore_axis_name="s",
        num_cores=NC, num_subcores=NS)


def _deg_body(colp2, wp2, degp, col_v, w_v, zero_v, acc, sem):
    c = lax.axis_index("c")
    s = lax.axis_index("s")

    @pl.when(s == 0)
    def _zero():
        def zf(i, _):
            zero_v[pl.ds(i * 16, 16)] = jnp.zeros((16,), f32)
            return 0
        lax.fori_loop(0, 200, zf, 0)
        for j in range(NS):
            pltpu.sync_copy(zero_v, acc.at[pl.ds(j * 3200, 3200)])

    plsc.subcore_barrier()

    n_chunks = EP // (NC * NS * K)
    base_r0 = (c * NS + s) * (EP // (NC * NS * SUB))

    def chunk(i, _):
        br = base_r0 + i * NSUB
        pltpu.sync_copy(colp2.at[pl.ds(br, NSUB)], col_v)
        pltpu.sync_copy(wp2.at[pl.ds(br, NSUB)], w_v)
        for b in range(NSUB):
            pltpu.sync_copy(w_v.at[b], acc.at[col_v.at[b]], add=True)
        return 0

    lax.fori_loop(0, n_chunks, chunk, 0)
    plsc.subcore_barrier()

    @pl.when(s == 0)
    def _write():
        pltpu.sync_copy(acc, degp.at[pl.ds(c * NZ, NZ)])


def _build_deg():
    return pl.kernel(
        _deg_body,
        out_type=jax.ShapeDtypeStruct((NC * NZ,), f32),
        mesh=_mesh(),
        compiler_params=pltpu.CompilerParams(needs_layout_passes=False, use_tc_tiling_on_sc=False),
        scratch_types=[
            pltpu.VMEM((NSUB, SUB), i32),
            pltpu.VMEM((NSUB, SUB), f32),
            pltpu.VMEM((3200,), f32),
            pltpu.VMEM_SHARED((NZ,), f32),
            pltpu.SemaphoreType.DMA,
        ],
    )


def _norm_body(rowp2, colp2, wp2, dinv, normp2, row_v, col_v, w_v, dinv_v, sem):
    c = lax.axis_index("c")
    s = lax.axis_index("s")
    pltpu.sync_copy(dinv, dinv_v)

    n_chunks = EP // (NC * NS * K)
    base_r0 = (c * NS + s) * (EP // (NC * NS * SUB))

    def chunk(i, _):
        br = base_r0 + i * NSUB
        pltpu.sync_copy(rowp2.at[pl.ds(br, NSUB)], row_v)
        pltpu.sync_copy(colp2.at[pl.ds(br, NSUB)], col_v)
        pltpu.sync_copy(wp2.at[pl.ds(br, NSUB)], w_v)
        for b in range(NSUB):
            def g16(g, _, b=b):
                sl = pl.ds(g * 16, 16)
                r16 = row_v[b, sl]
                c16 = col_v[b, sl]
                dr = plsc.load_gather(dinv_v, [r16 >> 7, r16 & 127])
                dc = plsc.load_gather(dinv_v, [c16 >> 7, c16 & 127])
                w_v[b, sl] = dr * w_v[b, sl] * dc
                return 0
            lax.fori_loop(0, SUB // 16, g16, 0)
        pltpu.sync_copy(w_v, normp2.at[pl.ds(br, NSUB)])
        return 0

    lax.fori_loop(0, n_chunks, chunk, 0)


def _build_norm():
    return pl.kernel(
        _norm_body,
        out_type=jax.ShapeDtypeStruct((EP // SUB, SUB), f32),
        mesh=_mesh(),
        compiler_params=pltpu.CompilerParams(needs_layout_passes=False, use_tc_tiling_on_sc=False),
        scratch_types=[
            pltpu.VMEM((NSUB, SUB), i32),
            pltpu.VMEM((NSUB, SUB), i32),
            pltpu.VMEM((NSUB, SUB), f32),
            pltpu.VMEM((NZ // 128, 128), f32),
            pltpu.SemaphoreType.DMA,
        ],
    )


def _make_spmm(feature_split, boff=0):
    H = 16
    if feature_split:
        n_chunks = EP // (NS * K2)
    else:
        n_chunks = EP // (NC * NS * K2)

    qb = boff // N

    def body(table, rows4, colp2, normp2, out,
             row_v, col_v, norm_v, rows_v, msg_v, zero_v, acc, sem0, sem1):
        c = lax.axis_index("c")
        s = lax.axis_index("s")

        def zf(i, _):
            zero_v[i, pl.ds(0, 16)] = jnp.zeros((16,), f32)
            return 0
        lax.fori_loop(0, ZB, zf, 0)
        for j in range(RPT // ZB):
            pltpu.sync_copy(zero_v, acc.at[pl.ds(s * RPT + j * ZB, ZB)])
        plsc.subcore_barrier()

        if feature_split:
            base_r0 = s * (EP // (NS * SUB))
        else:
            base_r0 = (c * NS + s) * (EP // (NC * NS * SUB))

        sems = (sem0, sem1)

        def fire(p, i):
            br = base_r0 + i * NSUB2
            pltpu.sync_copy(rows4.at[qb + c, pl.ds(br, NSUB2)], row_v.at[p])
            pltpu.sync_copy(colp2.at[pl.ds(br, NSUB2)], col_v.at[p])
            pltpu.sync_copy(normp2.at[pl.ds(br, NSUB2)], norm_v.at[p])
            for b in range(NSUB2):
                pltpu.async_copy(table.at[row_v.at[p, b]],
                                 rows_v.at[p, pl.ds(b * SUB, SUB)], sems[p])

        def drain(p):
            for b in range(NSUB2):
                pltpu.make_async_copy(table.at[row_v.at[p, b]],
                                      rows_v.at[p, pl.ds(b * SUB, SUB)],
                                      sems[p]).wait()

        def process(p):
            for b in range(NSUB2):
                @plsc.parallel_loop(0, SUB // 16, 1, unroll=2)
                def _scale(g, b=b):
                    n16 = norm_v[p, b, pl.ds(g * 16, 16)]
                    r16 = lax.iota(i32, 16) + (b * SUB + g * 16)
                    for t in range(H):
                        cvec = jnp.full((16,), t, i32)
                        v = plsc.load_gather(rows_v.at[p], [r16, cvec])
                        plsc.store_scatter(msg_v.at[p], [r16, cvec], v * n16)
            for b in range(NSUB2):
                pltpu.sync_copy(msg_v.at[p, pl.ds(b * SUB, SUB)],
                                acc.at[col_v.at[p, b]], add=True)

        npairs = n_chunks // 2
        fire(0, 0)

        def pair(j, _):
            drain(0)
            fire(1, 2 * j + 1)
            process(0)
            drain(1)

            @pl.when(j < npairs - 1)
            def _next():
                fire(0, 2 * j + 2)
            process(1)
            return 0

        lax.fori_loop(0, npairs, pair, 0)
        plsc.subcore_barrier()
        pltpu.sync_copy(acc.at[pl.ds(s * RPT, RPT)],
                        out.at[pl.ds(c * NP + s * RPT, RPT)])

    return pl.kernel(
        body,
        out_type=jax.ShapeDtypeStruct((NC * NP, H), f32),
        mesh=_mesh(),
        compiler_params=pltpu.CompilerParams(needs_layout_passes=False, use_tc_tiling_on_sc=False),
        scratch_types=[
            pltpu.VMEM((2, NSUB2, SUB), i32),
            pltpu.VMEM((2, NSUB2, SUB), i32),
            pltpu.VMEM((2, NSUB2, SUB), f32),
            pltpu.VMEM((2, K2, H), f32),
            pltpu.VMEM((2, K2, H), f32),
            pltpu.VMEM((ZB, H), f32),
            pltpu.VMEM_SHARED((NP, H), f32),
            pltpu.SemaphoreType.DMA,
            pltpu.SemaphoreType.DMA,
        ],
    )


_sc_cache = {}


def _sc(name):
    if name not in _sc_cache:
        _sc_cache["deg"] = _build_deg()
        _sc_cache["norm"] = _build_norm()
        _sc_cache["spmm16f"] = _make_spmm(True)
        _sc_cache["spmm16fb"] = _make_spmm(True, boff=2 * N)
        _sc_cache["spmm16e"] = _make_spmm(False)
    return _sc_cache[name]


BN = 2000
G = N // BN


def _spec(kind, d=0):
    if kind == "r":
        return pl.BlockSpec((BN, d), lambda i: (i, 0))
    if kind == "h":
        return pl.BlockSpec((2, BN, d), lambda i: (0, i, 0))
    if kind == "h4":
        return pl.BlockSpec((4, BN, d), lambda i: (0, i, 0))
    if kind == "w":
        return pl.BlockSpec(d, lambda i: tuple(0 for _ in d))
    raise ValueError(kind)


def _dinv_body(degp, dinv):
    dp = degp[...]
    dinv[...] = lax.rsqrt(dp[0] + dp[1])


def _dinv_call(degp, interpret=False):
    out = pl.pallas_call(
        _dinv_body,
        out_shape=jax.ShapeDtypeStruct((20, 2560), f32),
        interpret=interpret,
    )(degp.reshape(2, 20, 2560))
    return out.reshape(NZ)


def _tc(body, ins, in_specs, out_shapes, out_specs, interpret=False):
    return pl.pallas_call(
        body,
        grid=(G,),
        in_specs=in_specs,
        out_specs=out_specs,
        out_shape=out_shapes,
        interpret=interpret,
    )(*ins)


def _tc1_body(x, y, we1, h1):
    theta = jnp.concatenate([x[...], y[...]], axis=1)
    h = jnp.dot(theta, we1[...], preferred_element_type=f32)
    h1[0] = h[:, :16]
    h1[1] = h[:, 16:]


def _tc2_body(s1, b1, we2, h2):
    theta = jnp.maximum(
        jnp.concatenate([s1[0], s1[1]], axis=1) + b1[...], 0.0)
    h = jnp.dot(theta, we2[...], preferred_element_type=f32)
    for q in range(4):
        h2[q] = h[:, q * 16:(q + 1) * 16]


def _tc3_body(s2a, s2b, b2, wml, h3):
    theta = jnp.maximum(
        jnp.concatenate([s2a[0], s2a[1], s2b[0], s2b[1]], axis=1)
        + b2[...], 0.0)
    h = jnp.dot(theta, wml[...], preferred_element_type=f32)
    h3[0] = h
    h3[1] = h


def _tc4_body(s3, x, eps, bml, wg0, ws0, h4, ys0):
    ml = s3[0] + s3[1] + bml[...]
    mu = ml[:, 0:1]
    logvar = ml[:, 1:2]
    z = mu + eps[...] * jnp.exp(0.5 * logvar)
    recon = jnp.concatenate([z, x[...]], axis=1)
    h = jnp.dot(recon, wg0[...], preferred_element_type=f32)
    h4[0] = h[:, :16]
    h4[1] = h[:, 16:]
    ys0[...] = jnp.dot(recon, ws0[...], preferred_element_type=f32)


def _tc5_body(s4, ys0, x, bg0, wg1, ws1, h5, ys1):
    g = jnp.maximum(jnp.concatenate([s4[0], s4[1]], axis=1) + bg0[...], 0.0)
    yh = g + ys0[...]
    recon = jnp.concatenate([yh, x[...]], axis=1)
    h = jnp.dot(recon, wg1[...], preferred_element_type=f32)
    for q in range(4):
        h5[q] = h[:, q * 16:(q + 1) * 16]
    ys1[...] = jnp.dot(recon, ws1[...], preferred_element_type=f32)


def _tc6_body(s5a, s5b, ys1, x, bg1, wg2, ws2, h6, ys2):
    g = jnp.maximum(
        jnp.concatenate([s5a[0], s5a[1], s5b[0], s5b[1]], axis=1)
        + bg1[...], 0.0)
    yh = g + ys1[...]
    recon = jnp.concatenate([yh, x[...]], axis=1)
    h = jnp.dot(recon, wg2[...], preferred_element_type=f32)
    h6[0] = h
    h6[1] = h
    ys2[...] = jnp.dot(recon, ws2[...], preferred_element_type=f32)


def _tc7_body(s6, ys2, bg2, out):
    g = jnp.maximum(s6[0][:, :PRED] + s6[1][:, :PRED] + bg2[...], 0.0)
    out[...] = g + ys2[...]


def _run(x, y, edge_idx, edge_wt, params, interpret=False,
         deg_call=None, norm_call=None, spmm16f=None, spmm16fb=None,
         spmm16e=None):
    deg_call = deg_call or _sc("deg")
    norm_call = norm_call or _sc("norm")
    spmm16f = spmm16f or _sc("spmm16f")
    spmm16fb = spmm16fb or _sc("spmm16fb")
    spmm16e = spmm16e or _sc("spmm16e")
    p = params

    row = edge_idx[0]
    col = edge_idx[1]
    loops = jnp.arange(N, dtype=i32)
    padi = (jnp.arange(PAD, dtype=i32) * 11) % N
    rowp = jnp.concatenate([row, loops, padi]).reshape(EP // SUB, SUB)
    rows4 = rowp[None, :, :] + (jnp.arange(4, dtype=i32)[:, None, None] * N)
    colp = jnp.concatenate([col, loops, padi]).reshape(EP // SUB, SUB)
    wp = jnp.concatenate([
        edge_wt, jnp.full((N,), 2.0, f32), jnp.zeros((PAD,), f32)
    ]).reshape(EP // SUB, SUB)

    eps = jax.random.uniform(jax.random.key(42), (N, 1), dtype=f32)

    wml = jnp.pad(jnp.concatenate([p['W_mu'], p['W_var']], axis=1),
                  ((0, 0), (0, 14)))
    bml = jnp.pad(jnp.concatenate([p['b_mu'], p['b_var']]), (0, 14))
    wg2 = jnp.pad(p['W_g2'], ((0, 0), (0, 16 - PRED)))

    def b2d(b):
        return b.reshape(1, -1)

    degp = deg_call(colp, wp)
    dinv = _dinv_call(degp, interpret)
    h1 = _tc(
        _tc1_body,
        (x, y, p['W_e1']),
        [_spec("r", HIST), _spec("r", PRED), _spec("w", (24, 32))],
        jax.ShapeDtypeStruct((2, N, 16), f32),
        _spec("h", 16),
        interpret,
    )
    normp = norm_call(rowp, colp, wp, dinv.reshape(NZ // 128, 128))

    s1 = spmm16f(h1.reshape(2 * N, 16), rows4, colp, normp)
    h2 = _tc(
        _tc2_body,
        (s1.reshape(2, NP, 16), b2d(p['b_e1']), p['W_e2']),
        [_spec("h", 16), _spec("w", (1, 32)), _spec("w", (32, 64))],
        jax.ShapeDtypeStruct((4, N, 16), f32),
        _spec("h4", 16),
        interpret,
    )
    h2v = h2.reshape(4 * N, 16)
    s2a = spmm16f(h2v, rows4, colp, normp)
    s2b = spmm16fb(h2v, rows4, colp, normp)
    h3 = _tc(
        _tc3_body,
        (s2a.reshape(2, NP, 16), s2b.reshape(2, NP, 16), b2d(p['b_e2']), wml),
        [_spec("h", 16), _spec("h", 16), _spec("w", (1, 64)),
         _spec("w", (64, 16))],
        jax.ShapeDtypeStruct((2, N, 16), f32),
        _spec("h", 16),
        interpret,
    )
    s3 = spmm16e(h3.reshape(2 * N, 16), rows4, colp, normp)
    h4, ys0 = _tc(
        _tc4_body,
        (s3.reshape(2, NP, 16), x, eps, b2d(bml), p['W_g0'], p['W_s0']),
        [_spec("h", 16), _spec("r", HIST), _spec("r", 1), _spec("w", (1, 16)),
         _spec("w", (13, 32)), _spec("w", (13, 32))],
        (jax.ShapeDtypeStruct((2, N, 16), f32),
         jax.ShapeDtypeStruct((N, 32), f32)),
        (_spec("h", 16), _spec("r", 32)),
        interpret,
    )
    s4 = spmm16f(h4.reshape(2 * N, 16), rows4, colp, normp)
    h5, ys1 = _tc(
        _tc5_body,
        (s4.reshape(2, NP, 16), ys0, x, b2d(p['b_g0']), p['W_g1'], p['W_s1']),
        [_spec("h", 16), _spec("r", 32), _spec("r", HIST), _spec("w", (1, 32)),
         _spec("w", (44, 64)), _spec("w", (44, 64))],
        (jax.ShapeDtypeStruct((4, N, 16), f32),
         jax.ShapeDtypeStruct((N, 64), f32)),
        (_spec("h4", 16), _spec("r", 64)),
        interpret,
    )
    h5v = h5.reshape(4 * N, 16)
    s5a = spmm16f(h5v, rows4, colp, normp)
    s5b = spmm16fb(h5v, rows4, colp, normp)
    h6, ys2 = _tc(
        _tc6_body,
        (s5a.reshape(2, NP, 16), s5b.reshape(2, NP, 16), ys1, x,
         b2d(p['b_g1']), wg2, p['W_s2']),
        [_spec("h", 16), _spec("h", 16), _spec("r", 64), _spec("r", HIST),
         _spec("w", (1, 64)), _spec("w", (76, 16)), _spec("w", (76, PRED))],
        (jax.ShapeDtypeStruct((2, N, 16), f32),
         jax.ShapeDtypeStruct((N, PRED), f32)),
        (_spec("h", 16), _spec("r", PRED)),
        interpret,
    )
    s6 = spmm16e(h6.reshape(2 * N, 16), rows4, colp, normp)
    out = _tc(
        _tc7_body,
        (s6.reshape(2, NP, 16), ys2, b2d(p['b_g2'])),
        [_spec("h", 16), _spec("r", PRED), _spec("w", (1, PRED))],
        jax.ShapeDtypeStruct((N, PRED), f32),
        _spec("r", PRED),
        interpret,
    )
    return out


def kernel(x, y, edge_idx, edge_wt, params):
    return _run(x, y, edge_idx, edge_wt, params)


# speedup vs baseline: 1.8021x; 1.5267x over previous
import jax
import jax.numpy as jnp
from jax import lax
from jax.experimental import pallas as pl
from jax.experimental.pallas import tpu as pltpu
from jax.experimental.pallas import tpu_sc as plsc

N = 50000
E = 800000
HIST = 12
PRED = 12

NC = 2
NS = 16
SUB = 128
NSUB = 8
K = SUB * NSUB
NSUB2 = 8
K2 = SUB * NSUB2
EP = 851968
PAD = EP - E - N
NZ = 51200
NP = 50048
RPT = NP // NS
ZB = 136

f32 = jnp.float32
i32 = jnp.int32

def _mesh():
    return plsc.VectorSubcoreMesh(
        core_axis_name="c", subcore_axis_name="s",
        num_cores=NC, num_subcores=NS)


def _deg_body(colp2, wp2, degp, col_v, w_v, zero_v, acc, sem):
    c = lax.axis_index("c")
    s = lax.axis_index("s")

    @pl.when(s == 0)
    def _zero():
        def zf(i, _):
            zero_v[pl.ds(i * 16, 16)] = jnp.zeros((16,), f32)
            return 0
        lax.fori_loop(0, 200, zf, 0)
        for j in range(NS):
            pltpu.sync_copy(zero_v, acc.at[pl.ds(j * 3200, 3200)])

    plsc.subcore_barrier()

    n_chunks = EP // (NC * NS * K)
    base_r0 = (c * NS + s) * (EP // (NC * NS * SUB))

    def chunk(i, _):
        br = base_r0 + i * NSUB
        pltpu.sync_copy(colp2.at[pl.ds(br, NSUB)], col_v)
        pltpu.sync_copy(wp2.at[pl.ds(br, NSUB)], w_v)
        for b in range(NSUB):
            pltpu.sync_copy(w_v.at[b], acc.at[col_v.at[b]], add=True)
        return 0

    lax.fori_loop(0, n_chunks, chunk, 0)
    plsc.subcore_barrier()

    @pl.when(s == 0)
    def _write():
        pltpu.sync_copy(acc, degp.at[pl.ds(c * NZ, NZ)])


def _build_deg():
    return pl.kernel(
        _deg_body,
        out_type=jax.ShapeDtypeStruct((NC * NZ,), f32),
        mesh=_mesh(),
        compiler_params=pltpu.CompilerParams(needs_layout_passes=False, use_tc_tiling_on_sc=False),
        scratch_types=[
            pltpu.VMEM((NSUB, SUB), i32),
            pltpu.VMEM((NSUB, SUB), f32),
            pltpu.VMEM((3200,), f32),
            pltpu.VMEM_SHARED((NZ,), f32),
            pltpu.SemaphoreType.DMA,
        ],
    )


def _norm_body(rowp2, colp2, wp2, dinv, normp2, row_v, col_v, w_v, dinv_v, sem):
    c = lax.axis_index("c")
    s = lax.axis_index("s")
    pltpu.sync_copy(dinv, dinv_v)

    n_chunks = EP // (NC * NS * K)
    base_r0 = (c * NS + s) * (EP // (NC * NS * SUB))

    def chunk(i, _):
        br = base_r0 + i * NSUB
        pltpu.sync_copy(rowp2.at[pl.ds(br, NSUB)], row_v)
        pltpu.sync_copy(colp2.at[pl.ds(br, NSUB)], col_v)
        pltpu.sync_copy(wp2.at[pl.ds(br, NSUB)], w_v)
        for b in range(NSUB):
            def g16(g, _, b=b):
                sl = pl.ds(g * 16, 16)
                r16 = row_v[b, sl]
                c16 = col_v[b, sl]
                dr = plsc.load_gather(dinv_v, [r16 >> 7, r16 & 127])
                dc = plsc.load_gather(dinv_v, [c16 >> 7, c16 & 127])
                w_v[b, sl] = dr * w_v[b, sl] * dc
                return 0
            lax.fori_loop(0, SUB // 16, g16, 0)
        pltpu.sync_copy(w_v, normp2.at[pl.ds(br, NSUB)])
        return 0

    lax.fori_loop(0, n_chunks, chunk, 0)


def _build_norm():
    return pl.kernel(
        _norm_body,
        out_type=jax.ShapeDtypeStruct((EP // SUB, SUB), f32),
        mesh=_mesh(),
        compiler_params=pltpu.CompilerParams(needs_layout_passes=False, use_tc_tiling_on_sc=False),
        scratch_types=[
            pltpu.VMEM((NSUB, SUB), i32),
            pltpu.VMEM((NSUB, SUB), i32),
            pltpu.VMEM((NSUB, SUB), f32),
            pltpu.VMEM((NZ // 128, 128), f32),
            pltpu.SemaphoreType.DMA,
        ],
    )


def _make_spmm(feature_split, boff=0):
    H = 16
    if feature_split:
        n_chunks = EP // (NS * K2)
    else:
        n_chunks = EP // (NC * NS * K2)

    qb = boff // N

    def body(table, rows4, colp2, normp2, out,
             row_v, col_v, norm_v, rows_v, msg_v, zero_v, acc, sem0, sem1):
        c = lax.axis_index("c")
        s = lax.axis_index("s")

        def zf(i, _):
            zero_v[i, pl.ds(0, 16)] = jnp.zeros((16,), f32)
            return 0
        lax.fori_loop(0, ZB, zf, 0)
        for j in range(RPT // ZB):
            pltpu.sync_copy(zero_v, acc.at[pl.ds(s * RPT + j * ZB, ZB)])
        plsc.subcore_barrier()

        if feature_split:
            base_r0 = s * (EP // (NS * SUB))
        else:
            base_r0 = (c * NS + s) * (EP // (NC * NS * SUB))

        sems = (sem0, sem1)

        def fire(p, i):
            br = base_r0 + i * NSUB2
            pltpu.sync_copy(rows4.at[qb + c, pl.ds(br, NSUB2)], row_v.at[p])
            pltpu.sync_copy(colp2.at[pl.ds(br, NSUB2)], col_v.at[p])
            pltpu.sync_copy(normp2.at[pl.ds(br, NSUB2)], norm_v.at[p])
            for b in range(NSUB2):
                pltpu.async_copy(table.at[row_v.at[p, b]],
                                 rows_v.at[p, pl.ds(b * SUB, SUB)], sems[p])

        def drain(p):
            for b in range(NSUB2):
                pltpu.make_async_copy(table.at[row_v.at[p, b]],
                                      rows_v.at[p, pl.ds(b * SUB, SUB)],
                                      sems[p]).wait()

        def process(p):
            @plsc.parallel_loop(0, K2 // 16, 1, unroll=2)
            def _scale(g):
                n16 = norm_v[p, g >> 3, pl.ds((g & 7) * 16, 16)]
                for jj in range(16):
                    ns = n16.at[jnp.full((16,), jj, i32)].get(
                        mode='promise_in_bounds')
                    e = g * 16 + jj
                    msg_v[p, e, pl.ds(0, 16)] = (
                        rows_v[p, e, pl.ds(0, 16)] * ns)
            for b in range(NSUB2):
                pltpu.sync_copy(msg_v.at[p, pl.ds(b * SUB, SUB)],
                                acc.at[col_v.at[p, b]], add=True)

        npairs = n_chunks // 2
        fire(0, 0)

        def pair(j, _):
            drain(0)
            fire(1, 2 * j + 1)
            process(0)
            drain(1)

            @pl.when(j < npairs - 1)
            def _next():
                fire(0, 2 * j + 2)
            process(1)
            return 0

        lax.fori_loop(0, npairs, pair, 0)
        plsc.subcore_barrier()
        pltpu.sync_copy(acc.at[pl.ds(s * RPT, RPT)],
                        out.at[pl.ds(c * NP + s * RPT, RPT)])

    return pl.kernel(
        body,
        out_type=jax.ShapeDtypeStruct((NC * NP, H), f32),
        mesh=_mesh(),
        compiler_params=pltpu.CompilerParams(needs_layout_passes=False, use_tc_tiling_on_sc=False),
        scratch_types=[
            pltpu.VMEM((2, NSUB2, SUB), i32),
            pltpu.VMEM((2, NSUB2, SUB), i32),
            pltpu.VMEM((2, NSUB2, SUB), f32),
            pltpu.VMEM((2, K2, H), f32),
            pltpu.VMEM((2, K2, H), f32),
            pltpu.VMEM((ZB, H), f32),
            pltpu.VMEM_SHARED((NP, H), f32),
            pltpu.SemaphoreType.DMA,
            pltpu.SemaphoreType.DMA,
        ],
    )


_sc_cache = {}


def _sc(name):
    if name not in _sc_cache:
        _sc_cache["deg"] = _build_deg()
        _sc_cache["norm"] = _build_norm()
        _sc_cache["spmm16f"] = _make_spmm(True)
        _sc_cache["spmm16fb"] = _make_spmm(True, boff=2 * N)
        _sc_cache["spmm16e"] = _make_spmm(False)
    return _sc_cache[name]


BN = 2000
G = N // BN


def _spec(kind, d=0):
    if kind == "r":
        return pl.BlockSpec((BN, d), lambda i: (i, 0))
    if kind == "h":
        return pl.BlockSpec((2, BN, d), lambda i: (0, i, 0))
    if kind == "h4":
        return pl.BlockSpec((4, BN, d), lambda i: (0, i, 0))
    if kind == "w":
        return pl.BlockSpec(d, lambda i: tuple(0 for _ in d))
    raise ValueError(kind)


def _dinv_body(degp, dinv):
    dp = degp[...]
    dinv[...] = lax.rsqrt(dp[0] + dp[1])


def _dinv_call(degp, interpret=False):
    out = pl.pallas_call(
        _dinv_body,
        out_shape=jax.ShapeDtypeStruct((20, 2560), f32),
        interpret=interpret,
    )(degp.reshape(2, 20, 2560))
    return out.reshape(NZ)


def _tc(body, ins, in_specs, out_shapes, out_specs, interpret=False):
    return pl.pallas_call(
        body,
        grid=(G,),
        in_specs=in_specs,
        out_specs=out_specs,
        out_shape=out_shapes,
        interpret=interpret,
    )(*ins)


def _tc1_body(x, y, we1, h1):
    theta = jnp.concatenate([x[...], y[...]], axis=1)
    h = jnp.dot(theta, we1[...], preferred_element_type=f32)
    h1[0] = h[:, :16]
    h1[1] = h[:, 16:]


def _tc2_body(s1, b1, we2, h2):
    theta = jnp.maximum(
        jnp.concatenate([s1[0], s1[1]], axis=1) + b1[...], 0.0)
    h = jnp.dot(theta, we2[...], preferred_element_type=f32)
    for q in range(4):
        h2[q] = h[:, q * 16:(q + 1) * 16]


def _tc3_body(s2a, s2b, b2, wml, h3):
    theta = jnp.maximum(
        jnp.concatenate([s2a[0], s2a[1], s2b[0], s2b[1]], axis=1)
        + b2[...], 0.0)
    h = jnp.dot(theta, wml[...], preferred_element_type=f32)
    h3[0] = h
    h3[1] = h


def _tc4_body(s3, x, eps, bml, wg0, ws0, h4, ys0):
    ml = s3[0] + s3[1] + bml[...]
    mu = ml[:, 0:1]
    logvar = ml[:, 1:2]
    z = mu + eps[...] * jnp.exp(0.5 * logvar)
    recon = jnp.concatenate([z, x[...]], axis=1)
    h = jnp.dot(recon, wg0[...], preferred_element_type=f32)
    h4[0] = h[:, :16]
    h4[1] = h[:, 16:]
    ys0[...] = jnp.dot(recon, ws0[...], preferred_element_type=f32)


def _tc5_body(s4, ys0, x, bg0, wg1, ws1, h5, ys1):
    g = jnp.maximum(jnp.concatenate([s4[0], s4[1]], axis=1) + bg0[...], 0.0)
    yh = g + ys0[...]
    recon = jnp.concatenate([yh, x[...]], axis=1)
    h = jnp.dot(recon, wg1[...], preferred_element_type=f32)
    for q in range(4):
        h5[q] = h[:, q * 16:(q + 1) * 16]
    ys1[...] = jnp.dot(recon, ws1[...], preferred_element_type=f32)


def _tc6_body(s5a, s5b, ys1, x, bg1, wg2, ws2, h6, ys2):
    g = jnp.maximum(
        jnp.concatenate([s5a[0], s5a[1], s5b[0], s5b[1]], axis=1)
        + bg1[...], 0.0)
    yh = g + ys1[...]
    recon = jnp.concatenate([yh, x[...]], axis=1)
    h = jnp.dot(recon, wg2[...], preferred_element_type=f32)
    h6[0] = h
    h6[1] = h
    ys2[...] = jnp.dot(recon, ws2[...], preferred_element_type=f32)


def _tc7_body(s6, ys2, bg2, out):
    g = jnp.maximum(s6[0][:, :PRED] + s6[1][:, :PRED] + bg2[...], 0.0)
    out[...] = g + ys2[...]


def _run(x, y, edge_idx, edge_wt, params, interpret=False,
         deg_call=None, norm_call=None, spmm16f=None, spmm16fb=None,
         spmm16e=None):
    deg_call = deg_call or _sc("deg")
    norm_call = norm_call or _sc("norm")
    spmm16f = spmm16f or _sc("spmm16f")
    spmm16fb = spmm16fb or _sc("spmm16fb")
    spmm16e = spmm16e or _sc("spmm16e")
    p = params

    row = edge_idx[0]
    col = edge_idx[1]
    loops = jnp.arange(N, dtype=i32)
    padi = (jnp.arange(PAD, dtype=i32) * 11) % N
    rowp = jnp.concatenate([row, loops, padi]).reshape(EP // SUB, SUB)
    rows4 = rowp[None, :, :] + (jnp.arange(4, dtype=i32)[:, None, None] * N)
    colp = jnp.concatenate([col, loops, padi]).reshape(EP // SUB, SUB)
    wp = jnp.concatenate([
        edge_wt, jnp.full((N,), 2.0, f32), jnp.zeros((PAD,), f32)
    ]).reshape(EP // SUB, SUB)

    eps = jax.random.uniform(jax.random.key(42), (N, 1), dtype=f32)

    wml = jnp.pad(jnp.concatenate([p['W_mu'], p['W_var']], axis=1),
                  ((0, 0), (0, 14)))
    bml = jnp.pad(jnp.concatenate([p['b_mu'], p['b_var']]), (0, 14))
    wg2 = jnp.pad(p['W_g2'], ((0, 0), (0, 16 - PRED)))

    def b2d(b):
        return b.reshape(1, -1)

    degp = deg_call(colp, wp)
    dinv = _dinv_call(degp, interpret)
    h1 = _tc(
        _tc1_body,
        (x, y, p['W_e1']),
        [_spec("r", HIST), _spec("r", PRED), _spec("w", (24, 32))],
        jax.ShapeDtypeStruct((2, N, 16), f32),
        _spec("h", 16),
        interpret,
    )
    normp = norm_call(rowp, colp, wp, dinv.reshape(NZ // 128, 128))

    s1 = spmm16f(h1.reshape(2 * N, 16), rows4, colp, normp)
    h2 = _tc(
        _tc2_body,
        (s1.reshape(2, NP, 16), b2d(p['b_e1']), p['W_e2']),
        [_spec("h", 16), _spec("w", (1, 32)), _spec("w", (32, 64))],
        jax.ShapeDtypeStruct((4, N, 16), f32),
        _spec("h4", 16),
        interpret,
    )
    h2v = h2.reshape(4 * N, 16)
    s2a = spmm16f(h2v, rows4, colp, normp)
    s2b = spmm16fb(h2v, rows4, colp, normp)
    h3 = _tc(
        _tc3_body,
        (s2a.reshape(2, NP, 16), s2b.reshape(2, NP, 16), b2d(p['b_e2']), wml),
        [_spec("h", 16), _spec("h", 16), _spec("w", (1, 64)),
         _spec("w", (64, 16))],
        jax.ShapeDtypeStruct((2, N, 16), f32),
        _spec("h", 16),
        interpret,
    )
    s3 = spmm16e(h3.reshape(2 * N, 16), rows4, colp, normp)
    h4, ys0 = _tc(
        _tc4_body,
        (s3.reshape(2, NP, 16), x, eps, b2d(bml), p['W_g0'], p['W_s0']),
        [_spec("h", 16), _spec("r", HIST), _spec("r", 1), _spec("w", (1, 16)),
         _spec("w", (13, 32)), _spec("w", (13, 32))],
        (jax.ShapeDtypeStruct((2, N, 16), f32),
         jax.ShapeDtypeStruct((N, 32), f32)),
        (_spec("h", 16), _spec("r", 32)),
        interpret,
    )
    s4 = spmm16f(h4.reshape(2 * N, 16), rows4, colp, normp)
    h5, ys1 = _tc(
        _tc5_body,
        (s4.reshape(2, NP, 16), ys0, x, b2d(p['b_g0']), p['W_g1'], p['W_s1']),
        [_spec("h", 16), _spec("r", 32), _spec("r", HIST), _spec("w", (1, 32)),
         _spec("w", (44, 64)), _spec("w", (44, 64))],
        (jax.ShapeDtypeStruct((4, N, 16), f32),
         jax.ShapeDtypeStruct((N, 64), f32)),
        (_spec("h4", 16), _spec("r", 64)),
        interpret,
    )
    h5v = h5.reshape(4 * N, 16)
    s5a = spmm16f(h5v, rows4, colp, normp)
    s5b = spmm16fb(h5v, rows4, colp, normp)
    h6, ys2 = _tc(
        _tc6_body,
        (s5a.reshape(2, NP, 16), s5b.reshape(2, NP, 16), ys1, x,
         b2d(p['b_g1']), wg2, p['W_s2']),
        [_spec("h", 16), _spec("h", 16), _spec("r", 64), _spec("r", HIST),
         _spec("w", (1, 64)), _spec("w", (76, 16)), _spec("w", (76, PRED))],
        (jax.ShapeDtypeStruct((2, N, 16), f32),
         jax.ShapeDtypeStruct((N, PRED), f32)),
        (_spec("h", 16), _spec("r", PRED)),
        interpret,
    )
    s6 = spmm16e(h6.reshape(2 * N, 16), rows4, colp, normp)
    out = _tc(
        _tc7_body,
        (s6.reshape(2, NP, 16), ys2, b2d(p['b_g2'])),
        [_spec("h", 16), _spec("r", PRED), _spec("w", (1, PRED))],
        jax.ShapeDtypeStruct((N, PRED), f32),
        _spec("r", PRED),
        interpret,
    )
    return out


def kernel(x, y, edge_idx, edge_wt, params):
    return _run(x, y, edge_idx, edge_wt, params)


# speedup vs baseline: 2.0477x; 1.1363x over previous
import jax
import jax.numpy as jnp
from jax import lax
from jax.experimental import pallas as pl
from jax.experimental.pallas import tpu as pltpu
from jax.experimental.pallas import tpu_sc as plsc

N = 50000
E = 800000
HIST = 12
PRED = 12

NC = 2
NS = 16
SUB = 128
NSUB = 8
K = SUB * NSUB
NSUB2 = 8
K2 = SUB * NSUB2
EP = 851968
PAD = EP - E - N
NZ = 51200
NP = 50048
RPT = NP // NS
ZB = 136

f32 = jnp.float32
i32 = jnp.int32

def _mesh():
    return plsc.VectorSubcoreMesh(
        core_axis_name="c", subcore_axis_name="s",
        num_cores=NC, num_subcores=NS)


def _deg_body(colp2, wp2, degp, col_v, w_v, zero_v, acc, sem):
    c = lax.axis_index("c")
    s = lax.axis_index("s")

    @pl.when(s == 0)
    def _zero():
        def zf(i, _):
            zero_v[pl.ds(i * 16, 16)] = jnp.zeros((16,), f32)
            return 0
        lax.fori_loop(0, 200, zf, 0)
        for j in range(NS):
            pltpu.sync_copy(zero_v, acc.at[pl.ds(j * 3200, 3200)])

    plsc.subcore_barrier()

    n_chunks = EP // (NC * NS * K)
    base_r0 = (c * NS + s) * (EP // (NC * NS * SUB))

    def chunk(i, _):
        br = base_r0 + i * NSUB
        pltpu.sync_copy(colp2.at[pl.ds(br, NSUB)], col_v)
        pltpu.sync_copy(wp2.at[pl.ds(br, NSUB)], w_v)
        for b in range(NSUB):
            pltpu.sync_copy(w_v.at[b], acc.at[col_v.at[b]], add=True)
        return 0

    lax.fori_loop(0, n_chunks, chunk, 0)
    plsc.subcore_barrier()

    @pl.when(s == 0)
    def _write():
        pltpu.sync_copy(acc, degp.at[pl.ds(c * NZ, NZ)])


def _build_deg():
    return pl.kernel(
        _deg_body,
        out_type=jax.ShapeDtypeStruct((NC * NZ,), f32),
        mesh=_mesh(),
        compiler_params=pltpu.CompilerParams(needs_layout_passes=False, use_tc_tiling_on_sc=False),
        scratch_types=[
            pltpu.VMEM((NSUB, SUB), i32),
            pltpu.VMEM((NSUB, SUB), f32),
            pltpu.VMEM((3200,), f32),
            pltpu.VMEM_SHARED((NZ,), f32),
            pltpu.SemaphoreType.DMA,
        ],
    )


def _norm_body(rowp2, colp2, wp2, dinv, comb, row_v, col_v, w_v, comb_v,
               dinv_v, sem):
    c = lax.axis_index("c")
    s = lax.axis_index("s")
    pltpu.sync_copy(dinv, dinv_v)

    n_chunks = EP // (NC * NS * K)
    base_r0 = (c * NS + s) * (EP // (NC * NS * SUB))

    def chunk(i, _):
        br = base_r0 + i * NSUB
        pltpu.sync_copy(rowp2.at[pl.ds(br, NSUB)], row_v)
        pltpu.sync_copy(colp2.at[pl.ds(br, NSUB)], col_v)
        pltpu.sync_copy(wp2.at[pl.ds(br, NSUB)], w_v)
        for b in range(NSUB):
            def g16(g, _, b=b):
                sl = pl.ds(g * 16, 16)
                r16 = row_v[b, sl]
                c16 = col_v[b, sl]
                dr = plsc.load_gather(dinv_v, [r16 >> 7, r16 & 127])
                dc = plsc.load_gather(dinv_v, [c16 >> 7, c16 & 127])
                nrm = dr * w_v[b, sl] * dc
                for q in range(4):
                    comb_v[b, q, sl] = r16 + (q * N)
                comb_v[b, 4, sl] = c16
                comb_v[b, 5, sl] = plsc.bitcast(nrm, i32)
                return 0
            lax.fori_loop(0, SUB // 16, g16, 0)
        pltpu.sync_copy(comb_v, comb.at[pl.ds(br, NSUB)])
        return 0

    lax.fori_loop(0, n_chunks, chunk, 0)


def _build_norm():
    return pl.kernel(
        _norm_body,
        out_type=jax.ShapeDtypeStruct((EP // SUB, 6, SUB), i32),
        mesh=_mesh(),
        compiler_params=pltpu.CompilerParams(needs_layout_passes=False, use_tc_tiling_on_sc=False),
        scratch_types=[
            pltpu.VMEM((NSUB, SUB), i32),
            pltpu.VMEM((NSUB, SUB), i32),
            pltpu.VMEM((NSUB, SUB), f32),
            pltpu.VMEM((NSUB, 6, SUB), i32),
            pltpu.VMEM((NZ // 128, 128), f32),
            pltpu.SemaphoreType.DMA,
        ],
    )


def _make_spmm(feature_split, boff=0):
    H = 16
    if feature_split:
        n_chunks = EP // (NS * K2)
    else:
        n_chunks = EP // (NC * NS * K2)

    qb = boff // N

    def body(table, comb, out,
             comb_v, rows_v, msg_v, zero_v, acc, sem0, sem1):
        c = lax.axis_index("c")
        s = lax.axis_index("s")

        def zf(i, _):
            zero_v[i, pl.ds(0, 16)] = jnp.zeros((16,), f32)
            return 0
        lax.fori_loop(0, ZB, zf, 0)
        for j in range(RPT // ZB):
            pltpu.sync_copy(zero_v, acc.at[pl.ds(s * RPT + j * ZB, ZB)])
        plsc.subcore_barrier()

        if feature_split:
            base_r0 = s * (EP // (NS * SUB))
        else:
            base_r0 = (c * NS + s) * (EP // (NC * NS * SUB))

        sems = (sem0, sem1)

        qc = qb + c

        def fire(p, i):
            br = base_r0 + i * NSUB2
            pltpu.sync_copy(comb.at[pl.ds(br, NSUB2)], comb_v.at[p])
            for b in range(NSUB2):
                pltpu.async_copy(table.at[comb_v.at[p, b, qc]],
                                 rows_v.at[p, pl.ds(b * SUB, SUB)], sems[p])

        def drain(p):
            for b in range(NSUB2):
                pltpu.make_async_copy(table.at[comb_v.at[p, b, qc]],
                                      rows_v.at[p, pl.ds(b * SUB, SUB)],
                                      sems[p]).wait()

        def process(p):
            @plsc.parallel_loop(0, K2 // 16, 1, unroll=2)
            def _scale(g):
                n16 = plsc.bitcast(
                    comb_v[p, g >> 3, 5, pl.ds((g & 7) * 16, 16)], f32)
                for jj in range(16):
                    ns = n16.at[jnp.full((16,), jj, i32)].get(
                        mode='promise_in_bounds')
                    e = g * 16 + jj
                    msg_v[p, e, pl.ds(0, 16)] = (
                        rows_v[p, e, pl.ds(0, 16)] * ns)
            for b in range(NSUB2):
                pltpu.sync_copy(msg_v.at[p, pl.ds(b * SUB, SUB)],
                                acc.at[comb_v.at[p, b, 4]], add=True)

        npairs = n_chunks // 2
        fire(0, 0)

        def pair(j, _):
            drain(0)
            fire(1, 2 * j + 1)
            process(0)
            drain(1)

            @pl.when(j < npairs - 1)
            def _next():
                fire(0, 2 * j + 2)
            process(1)
            return 0

        lax.fori_loop(0, npairs, pair, 0)
        plsc.subcore_barrier()
        pltpu.sync_copy(acc.at[pl.ds(s * RPT, RPT)],
                        out.at[pl.ds(c * NP + s * RPT, RPT)])

    return pl.kernel(
        body,
        out_type=jax.ShapeDtypeStruct((NC * NP, H), f32),
        mesh=_mesh(),
        compiler_params=pltpu.CompilerParams(needs_layout_passes=False, use_tc_tiling_on_sc=False),
        scratch_types=[
            pltpu.VMEM((2, NSUB2, 6, SUB), i32),
            pltpu.VMEM((2, K2, H), f32),
            pltpu.VMEM((2, K2, H), f32),
            pltpu.VMEM((ZB, H), f32),
            pltpu.VMEM_SHARED((NP, H), f32),
            pltpu.SemaphoreType.DMA,
            pltpu.SemaphoreType.DMA,
        ],
    )


_sc_cache = {}


def _sc(name):
    if name not in _sc_cache:
        _sc_cache["deg"] = _build_deg()
        _sc_cache["norm"] = _build_norm()
        _sc_cache["spmm16f"] = _make_spmm(True)
        _sc_cache["spmm16fb"] = _make_spmm(True, boff=2 * N)
        _sc_cache["spmm16e"] = _make_spmm(False)
    return _sc_cache[name]


BN = 2000
G = N // BN


def _spec(kind, d=0):
    if kind == "r":
        return pl.BlockSpec((BN, d), lambda i: (i, 0))
    if kind == "h":
        return pl.BlockSpec((2, BN, d), lambda i: (0, i, 0))
    if kind == "h4":
        return pl.BlockSpec((4, BN, d), lambda i: (0, i, 0))
    if kind == "w":
        return pl.BlockSpec(d, lambda i: tuple(0 for _ in d))
    raise ValueError(kind)


def _dinv_body(degp, dinv):
    dp = degp[...]
    dinv[...] = lax.rsqrt(dp[0] + dp[1])


def _dinv_call(degp, interpret=False):
    out = pl.pallas_call(
        _dinv_body,
        out_shape=jax.ShapeDtypeStruct((20, 2560), f32),
        interpret=interpret,
    )(degp.reshape(2, 20, 2560))
    return out.reshape(NZ)


def _tc(body, ins, in_specs, out_shapes, out_specs, interpret=False):
    return pl.pallas_call(
        body,
        grid=(G,),
        in_specs=in_specs,
        out_specs=out_specs,
        out_shape=out_shapes,
        interpret=interpret,
    )(*ins)


def _tc1_body(x, y, we1, h1):
    theta = jnp.concatenate([x[...], y[...]], axis=1)
    h = jnp.dot(theta, we1[...], preferred_element_type=f32)
    h1[0] = h[:, :16]
    h1[1] = h[:, 16:]


def _tc2_body(s1, b1, we2, h2):
    theta = jnp.maximum(
        jnp.concatenate([s1[0], s1[1]], axis=1) + b1[...], 0.0)
    h = jnp.dot(theta, we2[...], preferred_element_type=f32)
    for q in range(4):
        h2[q] = h[:, q * 16:(q + 1) * 16]


def _tc3_body(s2a, s2b, b2, wml, h3):
    theta = jnp.maximum(
        jnp.concatenate([s2a[0], s2a[1], s2b[0], s2b[1]], axis=1)
        + b2[...], 0.0)
    h = jnp.dot(theta, wml[...], preferred_element_type=f32)
    h3[0] = h
    h3[1] = h


def _tc4_body(s3, x, eps, bml, wg0, ws0, h4, ys0):
    ml = s3[0] + s3[1] + bml[...]
    mu = ml[:, 0:1]
    logvar = ml[:, 1:2]
    z = mu + eps[...] * jnp.exp(0.5 * logvar)
    recon = jnp.concatenate([z, x[...]], axis=1)
    h = jnp.dot(recon, wg0[...], preferred_element_type=f32)
    h4[0] = h[:, :16]
    h4[1] = h[:, 16:]
    ys0[...] = jnp.dot(recon, ws0[...], preferred_element_type=f32)


def _tc5_body(s4, ys0, x, bg0, wg1, ws1, h5, ys1):
    g = jnp.maximum(jnp.concatenate([s4[0], s4[1]], axis=1) + bg0[...], 0.0)
    yh = g + ys0[...]
    recon = jnp.concatenate([yh, x[...]], axis=1)
    h = jnp.dot(recon, wg1[...], preferred_element_type=f32)
    for q in range(4):
        h5[q] = h[:, q * 16:(q + 1) * 16]
    ys1[...] = jnp.dot(recon, ws1[...], preferred_element_type=f32)


def _tc6_body(s5a, s5b, ys1, x, bg1, wg2, ws2, h6, ys2):
    g = jnp.maximum(
        jnp.concatenate([s5a[0], s5a[1], s5b[0], s5b[1]], axis=1)
        + bg1[...], 0.0)
    yh = g + ys1[...]
    recon = jnp.concatenate([yh, x[...]], axis=1)
    h = jnp.dot(recon, wg2[...], preferred_element_type=f32)
    h6[0] = h
    h6[1] = h
    ys2[...] = jnp.dot(recon, ws2[...], preferred_element_type=f32)


def _tc7_body(s6, ys2, bg2, out):
    g = jnp.maximum(s6[0][:, :PRED] + s6[1][:, :PRED] + bg2[...], 0.0)
    out[...] = g + ys2[...]


def _run(x, y, edge_idx, edge_wt, params, interpret=False,
         deg_call=None, norm_call=None, spmm16f=None, spmm16fb=None,
         spmm16e=None):
    deg_call = deg_call or _sc("deg")
    norm_call = norm_call or _sc("norm")
    spmm16f = spmm16f or _sc("spmm16f")
    spmm16fb = spmm16fb or _sc("spmm16fb")
    spmm16e = spmm16e or _sc("spmm16e")
    p = params

    row = edge_idx[0]
    col = edge_idx[1]
    loops = jnp.arange(N, dtype=i32)
    padi = (jnp.arange(PAD, dtype=i32) * 11) % N
    rowp = jnp.concatenate([row, loops, padi]).reshape(EP // SUB, SUB)
    colp = jnp.concatenate([col, loops, padi]).reshape(EP // SUB, SUB)
    wp = jnp.concatenate([
        edge_wt, jnp.full((N,), 2.0, f32), jnp.zeros((PAD,), f32)
    ]).reshape(EP // SUB, SUB)

    eps = jax.random.uniform(jax.random.key(42), (N, 1), dtype=f32)

    wml = jnp.pad(jnp.concatenate([p['W_mu'], p['W_var']], axis=1),
                  ((0, 0), (0, 14)))
    bml = jnp.pad(jnp.concatenate([p['b_mu'], p['b_var']]), (0, 14))
    wg2 = jnp.pad(p['W_g2'], ((0, 0), (0, 16 - PRED)))

    def b2d(b):
        return b.reshape(1, -1)

    degp = deg_call(colp, wp)
    dinv = _dinv_call(degp, interpret)
    h1 = _tc(
        _tc1_body,
        (x, y, p['W_e1']),
        [_spec("r", HIST), _spec("r", PRED), _spec("w", (24, 32))],
        jax.ShapeDtypeStruct((2, N, 16), f32),
        _spec("h", 16),
        interpret,
    )
    comb = norm_call(rowp, colp, wp, dinv.reshape(NZ // 128, 128))

    s1 = spmm16f(h1.reshape(2 * N, 16), comb)
    h2 = _tc(
        _tc2_body,
        (s1.reshape(2, NP, 16), b2d(p['b_e1']), p['W_e2']),
        [_spec("h", 16), _spec("w", (1, 32)), _spec("w", (32, 64))],
        jax.ShapeDtypeStruct((4, N, 16), f32),
        _spec("h4", 16),
        interpret,
    )
    h2v = h2.reshape(4 * N, 16)
    s2a = spmm16f(h2v, comb)
    s2b = spmm16fb(h2v, comb)
    h3 = _tc(
        _tc3_body,
        (s2a.reshape(2, NP, 16), s2b.reshape(2, NP, 16), b2d(p['b_e2']), wml),
        [_spec("h", 16), _spec("h", 16), _spec("w", (1, 64)),
         _spec("w", (64, 16))],
        jax.ShapeDtypeStruct((2, N, 16), f32),
        _spec("h", 16),
        interpret,
    )
    s3 = spmm16e(h3.reshape(2 * N, 16), comb)
    h4, ys0 = _tc(
        _tc4_body,
        (s3.reshape(2, NP, 16), x, eps, b2d(bml), p['W_g0'], p['W_s0']),
        [_spec("h", 16), _spec("r", HIST), _spec("r", 1), _spec("w", (1, 16)),
         _spec("w", (13, 32)), _spec("w", (13, 32))],
        (jax.ShapeDtypeStruct((2, N, 16), f32),
         jax.ShapeDtypeStruct((N, 32), f32)),
        (_spec("h", 16), _spec("r", 32)),
        interpret,
    )
    s4 = spmm16f(h4.reshape(2 * N, 16), comb)
    h5, ys1 = _tc(
        _tc5_body,
        (s4.reshape(2, NP, 16), ys0, x, b2d(p['b_g0']), p['W_g1'], p['W_s1']),
        [_spec("h", 16), _spec("r", 32), _spec("r", HIST), _spec("w", (1, 32)),
         _spec("w", (44, 64)), _spec("w", (44, 64))],
        (jax.ShapeDtypeStruct((4, N, 16), f32),
         jax.ShapeDtypeStruct((N, 64), f32)),
        (_spec("h4", 16), _spec("r", 64)),
        interpret,
    )
    h5v = h5.reshape(4 * N, 16)
    s5a = spmm16f(h5v, comb)
    s5b = spmm16fb(h5v, comb)
    h6, ys2 = _tc(
        _tc6_body,
        (s5a.reshape(2, NP, 16), s5b.reshape(2, NP, 16), ys1, x,
         b2d(p['b_g1']), wg2, p['W_s2']),
        [_spec("h", 16), _spec("h", 16), _spec("r", 64), _spec("r", HIST),
         _spec("w", (1, 64)), _spec("w", (76, 16)), _spec("w", (76, PRED))],
        (jax.ShapeDtypeStruct((2, N, 16), f32),
         jax.ShapeDtypeStruct((N, PRED), f32)),
        (_spec("h", 16), _spec("r", PRED)),
        interpret,
    )
    s6 = spmm16e(h6.reshape(2 * N, 16), comb)
    out = _tc(
        _tc7_body,
        (s6.reshape(2, NP, 16), ys2, b2d(p['b_g2'])),
        [_spec("h", 16), _spec("r", PRED), _spec("w", (1, PRED))],
        jax.ShapeDtypeStruct((N, PRED), f32),
        _spec("r", PRED),
        interpret,
    )
    return out


def kernel(x, y, edge_idx, edge_wt, params):
    return _run(x, y, edge_idx, edge_wt, params)
